# baseline (speedup 1.0000x reference)
import functools

import jax
import jax.numpy as jnp
from jax import lax
from jax.experimental import pallas as pl
from jax.experimental.pallas import tpu as pltpu
from jax.experimental.pallas import tpu_sc as plsc

EMBED_DIM = 32
CHUNK = 800
NBUF = 4


@functools.lru_cache(maxsize=None)
def _make(B: int, D: int):
    info = plsc.get_sparse_core_info()
    NC, NS = info.num_cores, info.num_subcores
    NW = NC * NS
    assert B % (NW * CHUNK * NBUF) == 0
    b_per_w = B // NW
    n_groups = b_per_w // (CHUNK * NBUF)
    mesh = plsc.VectorSubcoreMesh(core_axis_name="c", subcore_axis_name="s")

    scratch = (
        [pltpu.VMEM((CHUNK,), jnp.int32) for _ in range(NBUF)]
        + [pltpu.VMEM((CHUNK, D), jnp.float32) for _ in range(NBUF)]
        + [pltpu.SemaphoreType.DMA for _ in range(2 * NBUF)]
    )

    @functools.partial(
        pl.kernel,
        mesh=mesh,
        compiler_params=pltpu.CompilerParams(use_tc_tiling_on_sc=False),
        out_type=jax.ShapeDtypeStruct((B, D), jnp.float32),
        scratch_types=scratch,
    )
    def k(idx_hbm, table_hbm, out_hbm, *scr):
        idx_vs = scr[:NBUF]
        rows_vs = scr[NBUF : 2 * NBUF]
        gsems = scr[2 * NBUF : 3 * NBUF]
        osems = scr[3 * NBUF : 4 * NBUF]
        wid = lax.axis_index("s") * NC + lax.axis_index("c")
        base = wid * b_per_w

        def group(gi, carry):
            offs = [base + (gi * NBUF + b) * CHUNK for b in range(NBUF)]
            gathers = []
            for b in range(NBUF):
                @pl.when(gi > 0)
                def _drain(b=b):
                    pltpu.make_async_copy(
                        rows_vs[b], out_hbm.at[pl.ds(offs[b], CHUNK)], osems[b]
                    ).wait()

                pltpu.sync_copy(idx_hbm.at[pl.ds(offs[b], CHUNK)], idx_vs[b])
                gathers.append(
                    pltpu.async_copy(table_hbm.at[idx_vs[b]], rows_vs[b], gsems[b])
                )
            for b in range(NBUF):
                gathers[b].wait()
                pltpu.async_copy(
                    rows_vs[b], out_hbm.at[pl.ds(offs[b], CHUNK)], osems[b]
                )
            return carry

        lax.fori_loop(0, n_groups, group, 0)
        for b in range(NBUF):
            pltpu.make_async_copy(
                rows_vs[b], out_hbm.at[pl.ds(base + b * CHUNK, CHUNK)], osems[b]
            ).wait()

    return k


def kernel(x, table):
    B0, H = x.shape
    D = table.shape[1]
    idx = x.reshape(B0 * H).astype(jnp.int32)
    out = _make(B0 * H, D)(idx, table)
    return out.reshape(B0, H, D)


# speedup vs baseline: 1.3786x; 1.3786x over previous
import functools

import jax
import jax.numpy as jnp
from jax import lax
from jax.experimental import pallas as pl
from jax.experimental.pallas import tpu as pltpu
from jax.experimental.pallas import tpu_sc as plsc

EMBED_DIM = 32
CHUNK = 800
NBUF = 4


@functools.lru_cache(maxsize=None)
def _make(B: int, D: int, V: int):
    info = plsc.get_sparse_core_info()
    NC, NS = info.num_cores, info.num_subcores
    NW = NC * NS
    assert B % (NW * CHUNK * NBUF) == 0
    b_per_w = B // NW
    n_groups = b_per_w // (CHUNK * NBUF)
    mesh = plsc.VectorSubcoreMesh(core_axis_name="c", subcore_axis_name="s")

    scratch = (
        [pltpu.VMEM((CHUNK,), jnp.int32) for _ in range(NBUF)]
        + [pltpu.VMEM((CHUNK, D), jnp.float32) for _ in range(NBUF)]
        + [pltpu.SemaphoreType.DMA for _ in range(2 * NBUF)]
        + [pltpu.VMEM_SHARED((V, D), jnp.float32)]
    )

    @functools.partial(
        pl.kernel,
        mesh=mesh,
        compiler_params=pltpu.CompilerParams(use_tc_tiling_on_sc=False),
        out_type=jax.ShapeDtypeStruct((B, D), jnp.float32),
        scratch_types=scratch,
    )
    def k(idx_hbm, table_hbm, out_hbm, *scr):
        idx_vs = scr[:NBUF]
        rows_vs = scr[NBUF : 2 * NBUF]
        gsems = scr[2 * NBUF : 3 * NBUF]
        osems = scr[3 * NBUF : 4 * NBUF]
        table_sh = scr[4 * NBUF]
        sid = lax.axis_index("s")
        wid = sid * NC + lax.axis_index("c")
        base = wid * b_per_w

        @pl.when(sid == 0)
        def _stage():
            pltpu.sync_copy(table_hbm, table_sh)

        plsc.subcore_barrier()

        def group(gi, carry):
            offs = [base + (gi * NBUF + b) * CHUNK for b in range(NBUF)]
            gathers = []
            for b in range(NBUF):
                @pl.when(gi > 0)
                def _drain(b=b):
                    pltpu.make_async_copy(
                        rows_vs[b], out_hbm.at[pl.ds(offs[b], CHUNK)], osems[b]
                    ).wait()

                pltpu.sync_copy(idx_hbm.at[pl.ds(offs[b], CHUNK)], idx_vs[b])
                gathers.append(
                    pltpu.async_copy(table_sh.at[idx_vs[b]], rows_vs[b], gsems[b])
                )
            for b in range(NBUF):
                gathers[b].wait()
                pltpu.async_copy(
                    rows_vs[b], out_hbm.at[pl.ds(offs[b], CHUNK)], osems[b]
                )
            return carry

        lax.fori_loop(0, n_groups, group, 0)
        for b in range(NBUF):
            pltpu.make_async_copy(
                rows_vs[b], out_hbm.at[pl.ds(base + b * CHUNK, CHUNK)], osems[b]
            ).wait()

    return k


def kernel(x, table):
    B0, H = x.shape
    D = table.shape[1]
    idx = x.reshape(B0 * H).astype(jnp.int32)
    out = _make(B0 * H, D, table.shape[0])(idx, table)
    return out.reshape(B0, H, D)
